# Initial kernel scaffold; baseline (speedup 1.0000x reference)
import jax
import jax.numpy as jnp
from jax.experimental import pallas as pl


def kernel(x, edge_index, W1, b1, W2, b2, W_out, b_out):
    raise NotImplementedError("write your pallas kernel here")


# speedup vs baseline: 30.5181x; 30.5181x over previous
import functools

import jax
import jax.numpy as jnp
from jax import lax
from jax.experimental import pallas as pl
from jax.experimental.pallas import tpu as pltpu
from jax.experimental.pallas import tpu_sc as plsc

_N = 10000
_E = 320000
_NC = 2
_NS = 16
_NW = _NC * _NS
_B = 128
_K = -(-_E // (_NW * _B))
_EP = _NW * _K * _B
_NP = 10112
_RPW = _NP // _NS
_DEGW = 16

_mesh = plsc.VectorSubcoreMesh(core_axis_name="c", subcore_axis_name="s")


def _make_edge_pass(dcols):
  @functools.partial(
      pl.kernel,
      out_type=jax.ShapeDtypeStruct((_NC, _NP, dcols), jnp.float32),
      mesh=_mesh,
      scratch_types=[
          pltpu.VMEM((_K, _B), jnp.int32),
          pltpu.VMEM((_K, _B), jnp.int32),
          pltpu.VMEM((_B, dcols), jnp.float32),
          pltpu.VMEM((_RPW, dcols), jnp.float32),
          pltpu.VMEM_SHARED((_NP, dcols), jnp.float32),
          pltpu.SemaphoreType.DMA,
      ],
      compiler_params=pltpu.CompilerParams(use_tc_tiling_on_sc=False),
  )
  def edge_pass(g_hbm, src_hbm, dst_hbm, out_hbm,
                src_v, dst_v, rows_v, stage_v, acc_sh, sem):
    c = lax.axis_index("c")
    s = lax.axis_index("s")
    wid = c * _NS + s

    @pl.loop(0, _RPW)
    def _(i):
      for d0 in range(dcols // 16):
        stage_v[i, pl.ds(d0 * 16, 16)] = jnp.zeros((16,), jnp.float32)

    rows = pl.ds(s * _RPW, _RPW)
    pltpu.sync_copy(stage_v, acc_sh.at[rows])
    pltpu.sync_copy(src_hbm.at[wid], src_v)
    pltpu.sync_copy(dst_hbm.at[wid], dst_v)
    plsc.subcore_barrier()

    @pl.loop(0, _K)
    def _(j):
      pltpu.async_copy(g_hbm.at[src_v.at[j]], rows_v, sem).wait()
      pltpu.sync_copy(rows_v, acc_sh.at[dst_v.at[j]], add=True)

    plsc.subcore_barrier()
    pltpu.sync_copy(acc_sh.at[rows], out_hbm.at[c, rows])

  return edge_pass


_edge32 = _make_edge_pass(32)
_edge16 = _make_edge_pass(16)


@functools.partial(
    pl.kernel,
    out_type=jax.ShapeDtypeStruct((_NC, _NP, _DEGW), jnp.float32),
    mesh=_mesh,
    scratch_types=[
        pltpu.VMEM((_K, _B), jnp.int32),
        pltpu.VMEM((_B, _DEGW), jnp.float32),
        pltpu.VMEM((_RPW, _DEGW), jnp.float32),
        pltpu.VMEM_SHARED((_NP, _DEGW), jnp.float32),
    ],
    compiler_params=pltpu.CompilerParams(use_tc_tiling_on_sc=False),
)
def _deg_pass(dst_hbm, out_hbm, dst_v, ones_v, stage_v, acc_sh):
  c = lax.axis_index("c")
  s = lax.axis_index("s")
  wid = c * _NS + s

  @pl.loop(0, _RPW)
  def _(i):
    stage_v[i, pl.ds(0, 16)] = jnp.zeros((16,), jnp.float32)

  @pl.loop(0, _B)
  def _(i):
    ones_v[i, pl.ds(0, 16)] = jnp.ones((16,), jnp.float32)

  rows = pl.ds(s * _RPW, _RPW)
  pltpu.sync_copy(stage_v, acc_sh.at[rows])
  pltpu.sync_copy(dst_hbm.at[wid], dst_v)
  plsc.subcore_barrier()

  @pl.loop(0, _K)
  def _(j):
    pltpu.sync_copy(ones_v, acc_sh.at[dst_v.at[j]], add=True)

  plsc.subcore_barrier()
  pltpu.sync_copy(acc_sh.at[rows], out_hbm.at[c, rows])


def _mm_body(x_ref, w_ref, o_ref):
  o_ref[...] = jnp.dot(x_ref[...], w_ref[...],
                       preferred_element_type=jnp.float32)


def _scale_body(degt_ref, hw_ref, g_ref, dis_ref):
  deg = degt_ref[0, :, 0:1] + degt_ref[1, :, 0:1] + 1.0
  row = lax.broadcasted_iota(jnp.int32, (_NP, 1), 0)
  dis = jnp.where(row < _N, lax.rsqrt(deg), 0.0)
  dis_ref[...] = dis
  g_ref[...] = hw_ref[...] * dis


def _mid_body(acc_ref, g_ref, dis_ref, b_ref, w_ref, o_ref):
  agg = (acc_ref[0] + acc_ref[1] + g_ref[...]) * dis_ref[...] + b_ref[...]
  h = jnp.maximum(agg, 0.0)
  o_ref[...] = jnp.dot(h, w_ref[...],
                       preferred_element_type=jnp.float32) * dis_ref[...]


def _final_body(acc_ref, g_ref, dis_ref, b_ref, w_ref, bo_ref, o_ref):
  agg = (acc_ref[0] + acc_ref[1] + g_ref[...]) * dis_ref[...] + b_ref[...]
  h = jnp.maximum(agg, 0.0)
  o_ref[...] = jnp.dot(h, w_ref[...],
                       preferred_element_type=jnp.float32) + bo_ref[...]


def _f32(*shape):
  return jax.ShapeDtypeStruct(shape, jnp.float32)


def kernel(x, edge_index, W1, b1, W2, b2, W_out, b_out):
  xp = jnp.pad(x, ((0, _NP - _N), (0, 0)))
  pad = _EP - _E
  fill = jnp.full((pad,), _N, jnp.int32)
  srcp = jnp.concatenate([edge_index[0], fill]).reshape(_NW, _K, _B)
  dstp = jnp.concatenate([edge_index[1], fill]).reshape(_NW, _K, _B)

  hw1 = pl.pallas_call(_mm_body, out_shape=_f32(_NP, 32))(xp, W1)
  degt = _deg_pass(dstp)
  g1, dis = pl.pallas_call(
      _scale_body, out_shape=(_f32(_NP, 32), _f32(_NP, 1)))(degt, hw1)
  acc1 = _edge32(g1, srcp, dstp)
  g2 = pl.pallas_call(_mid_body, out_shape=_f32(_NP, 16))(
      acc1, g1, dis, b1.reshape(1, -1), W2)
  acc2 = _edge16(g2, srcp, dstp)
  y = pl.pallas_call(_final_body, out_shape=_f32(_NP, 1))(
      acc2, g2, dis, b2.reshape(1, -1), W_out, b_out.reshape(1, 1))
  return y[:_N]
